# baseline (speedup 1.0000x reference)
import functools

import jax
import jax.numpy as jnp
from jax import lax
from jax.experimental import pallas as pl
from jax.experimental.pallas import tpu as pltpu
from jax.experimental.pallas import tpu_sc as plsc

_N = 10000
_E = 320000
_D = 128
_DE = 16
_B = 64
_G = 32
_MV = 5
_L = 3


_BE = 2000


def _edge_mlp_body(attr_ref, w_ref, b_ref, o_ref):
    a = attr_ref[...].astype(jnp.bfloat16)
    e = jnp.dot(a, w_ref[...].astype(jnp.bfloat16),
                preferred_element_type=jnp.float32)
    o_ref[...] = jnp.maximum(e + b_ref[...], 0.0)


def _edge_mlp_one(edge_attr, w, b):
    grid = (_E // _BE,)
    return pl.pallas_call(
        _edge_mlp_body,
        grid=grid,
        in_specs=[
            pl.BlockSpec((_BE, _DE), lambda i: (i, 0)),
            pl.BlockSpec((_DE, _D), lambda i: (0, 0)),
            pl.BlockSpec((1, _D), lambda i: (0, 0)),
        ],
        out_specs=pl.BlockSpec((_BE, _D), lambda i: (i, 0)),
        out_shape=jax.ShapeDtypeStruct((_E, _D), jnp.float32),
    )(edge_attr, w, b)


_NC = 2
_NS = 16
_NW = _NC * _NS
_EPT = _E // _NW
_K = 40
_CHUNKS = _EPT // _K
_NB = 2
_GROUPS = _CHUNKS // _NB
_NP = 10240
_RPT = _NP // _NS
_ZR = 128


def _sc_msg_body(e_hbm, h_hbm, src_hbm, dst_hbm, out_hbm,
                 idxs_s, idxs_d, bufs_h, zbuf, agg,
                 sems_i, sems_e, sems_g, sems_o):
    c = lax.axis_index("c")
    s = lax.axis_index("s")
    wid = c * _NS + s

    def _zrow(i, carry):
        for j in range(_D // 16):
            zbuf[i, pl.ds(j * 16, 16)] = jnp.zeros((16,), jnp.float32)
        return carry

    lax.fori_loop(0, _ZR, _zrow, 0)
    r0 = s * _RPT
    for z in range(_RPT // _ZR):
        pltpu.sync_copy(zbuf, agg.at[pl.ds(r0 + z * _ZR, _ZR)])
    plsc.subcore_barrier()

    ebase = wid * _EPT

    def _group(g, carry):
        offs = [ebase + (g * _NB + j) * _K for j in range(_NB)]
        cps = []
        for j in range(_NB):
            cp_s = pltpu.async_copy(src_hbm.at[pl.ds(offs[j], _K)],
                                    idxs_s[j], sems_i[j])
            cp_d = pltpu.async_copy(dst_hbm.at[pl.ds(offs[j], _K)],
                                    idxs_d[j], sems_i[j])
            cp_e = pltpu.async_copy(e_hbm.at[pl.ds(offs[j], _K)],
                                    bufs_h[j], sems_e[j])
            cps.append((cp_s, cp_d, cp_e))
        gts = []
        for j in range(_NB):
            cps[j][0].wait()
            cps[j][2].wait()
            gts.append(pltpu.async_copy(h_hbm.at[idxs_s[j]], bufs_h[j],
                                        sems_g[j], add=True))
        scs = []
        for j in range(_NB):
            cps[j][1].wait()
            gts[j].wait()

            def _row(i, carry2):
                r = i * 2
                for dr in range(2):
                    for q in range(_D // 16):
                        sl = pl.ds(q * 16, 16)
                        bufs_h[j][r + dr, sl] = jnp.maximum(
                            bufs_h[j][r + dr, sl], 0.0)
                return carry2

            lax.fori_loop(0, _K // 2, _row, 0)
            scs.append(pltpu.async_copy(bufs_h[j], agg.at[idxs_d[j]],
                                        sems_o[j], add=True))
        for sc in scs:
            sc.wait()
        return carry

    lax.fori_loop(0, _GROUPS, _group, 0)
    plsc.subcore_barrier()

    for z in range(_RPT // _ZR):
        rr = s * _RPT + z * _ZR
        pltpu.sync_copy(agg.at[pl.ds(rr, _ZR)], out_hbm.at[c, pl.ds(rr, _ZR)])


@functools.cache
def _sc_msg_kernel():
    return pl.kernel(
        _sc_msg_body,
        out_type=jax.ShapeDtypeStruct((_NC, _NP, _D), jnp.float32),
        mesh=plsc.VectorSubcoreMesh(core_axis_name="c", subcore_axis_name="s"),
        scratch_types=[
            [pltpu.VMEM((_K,), jnp.int32)] * _NB,
            [pltpu.VMEM((_K,), jnp.int32)] * _NB,
            [pltpu.VMEM((_K, _D), jnp.float32)] * _NB,
            pltpu.VMEM((_ZR, _D), jnp.float32),
            pltpu.VMEM_SHARED((_NP, _D), jnp.float32),
            [pltpu.SemaphoreType.DMA] * _NB,
            [pltpu.SemaphoreType.DMA] * _NB,
            [pltpu.SemaphoreType.DMA] * _NB,
            [pltpu.SemaphoreType.DMA] * _NB,
        ],
    )


def _sc_msg(e, h, src, dst):
    return _sc_msg_kernel()(e, h, src, dst)


_BN = 1000


def _node_mlp_body(h_ref, agg_ref, eps_ref, w1_ref, b1_ref, w2_ref, b2_ref,
                   g_ref, bn_ref, out_ref):
    h = h_ref[...]
    t = (1.0 + eps_ref[0, 0]) * h + agg_ref[0] + agg_ref[1]
    u = jnp.dot(t.astype(jnp.bfloat16), w1_ref[...].astype(jnp.bfloat16),
                preferred_element_type=jnp.float32)
    u = jnp.maximum(u + b1_ref[...], 0.0)
    u = jnp.dot(u.astype(jnp.bfloat16), w2_ref[...].astype(jnp.bfloat16),
                preferred_element_type=jnp.float32)
    u = u + b2_ref[...]
    m = jnp.mean(u, axis=-1, keepdims=True)
    v = jnp.mean((u - m) * (u - m), axis=-1, keepdims=True)
    u = (u - m) * jax.lax.rsqrt(v + 1e-5) * g_ref[...] + bn_ref[...]
    out_ref[...] = jnp.maximum(u, 0.0) + h


def _node_mlp(h, agg, eps_i, w1, b1, w2, b2, g, bn):
    grid = (_N // _BN,)
    return pl.pallas_call(
        _node_mlp_body,
        grid=grid,
        in_specs=[
            pl.BlockSpec((_BN, _D), lambda i: (i, 0)),
            pl.BlockSpec((_NC, _BN, _D), lambda i: (0, i, 0)),
            pl.BlockSpec(memory_space=pltpu.SMEM),
            pl.BlockSpec((_D, _D), lambda i: (0, 0)),
            pl.BlockSpec((1, _D), lambda i: (0, 0)),
            pl.BlockSpec((_D, _D), lambda i: (0, 0)),
            pl.BlockSpec((1, _D), lambda i: (0, 0)),
            pl.BlockSpec((1, _D), lambda i: (0, 0)),
            pl.BlockSpec((1, _D), lambda i: (0, 0)),
        ],
        out_specs=pl.BlockSpec((_BN, _D), lambda i: (i, 0)),
        out_shape=jax.ShapeDtypeStruct((_N, _D), jnp.float32),
    )(h, agg, eps_i, w1, b1, w2, b2, g, bn)


def _pool_heads_body(h_ref, batch_ref, gf_ref, mv_ref,
                     pw1_ref, pb1_ref, pw2_ref, pb2_ref,
                     vw1_ref, vb1_ref, vw2_ref, vb2_ref,
                     pol_ref, val_ref, pooled_acc, cnt_acc):
    i = pl.program_id(0)

    @pl.when(i == 0)
    def _init():
        pooled_acc[...] = jnp.zeros_like(pooled_acc)
        cnt_acc[...] = jnp.zeros_like(cnt_acc)

    h = h_ref[...]
    b = batch_ref[...]
    oh = (b == lax.broadcasted_iota(jnp.int32, (_BN, _B), 1)).astype(jnp.float32)
    pooled_acc[...] += lax.dot_general(
        oh, h, (((0,), (0,)), ((), ())), preferred_element_type=jnp.float32,
        precision=jax.lax.Precision.HIGHEST)
    cnt_acc[...] += lax.dot_general(
        oh, jnp.ones((_BN, _D), jnp.float32), (((0,), (0,)), ((), ())),
        preferred_element_type=jnp.float32,
        precision=jax.lax.Precision.HIGHEST)

    @pl.when(i == pl.num_programs(0) - 1)
    def _heads():
        pooled = pooled_acc[...] / jnp.maximum(cnt_acc[...], 1.0)
        v_in = jnp.concatenate([pooled, gf_ref[...]], axis=1)
        v = jnp.maximum(
            jnp.dot(v_in.astype(jnp.bfloat16), vw1_ref[...].astype(jnp.bfloat16),
                    preferred_element_type=jnp.float32)
            + vb1_ref[...], 0.0)
        val_ref[...] = (jnp.dot(v.astype(jnp.bfloat16),
                                vw2_ref[...].astype(jnp.bfloat16),
                                preferred_element_type=jnp.float32)
                        + vb2_ref[...])
        p_in = jnp.concatenate([pooled, mv_ref[...]], axis=1)
        p = jnp.maximum(
            jnp.dot(p_in.astype(jnp.bfloat16), pw1_ref[...].astype(jnp.bfloat16),
                    preferred_element_type=jnp.float32)
            + pb1_ref[...], 0.0)
        pol_ref[...] = (jnp.dot(p.astype(jnp.bfloat16),
                                pw2_ref[...].astype(jnp.bfloat16),
                                preferred_element_type=jnp.float32)
                        + pb2_ref[...])


def _pool_heads(h, batch2d, gf, mv, pw1, pb1, pw2, pb2, vw1, vb1, vw2, vb2):
    grid = (_N // _BN,)
    full = lambda shape: pl.BlockSpec(shape, lambda i: tuple(0 for _ in shape))
    return pl.pallas_call(
        _pool_heads_body,
        grid=grid,
        in_specs=[
            pl.BlockSpec((_BN, _D), lambda i: (i, 0)),
            pl.BlockSpec((_BN, 1), lambda i: (i, 0)),
            full((_B, _G)),
            full((_B, _MV)),
            full((_D + _MV, _D)),
            full((1, _D)),
            full((_D, 1)),
            full((1, 1)),
            full((_D + _G, _D)),
            full((1, _D)),
            full((_D, 1)),
            full((1, 1)),
        ],
        out_specs=[full((_B, 1)), full((_B, 1))],
        out_shape=[jax.ShapeDtypeStruct((_B, 1), jnp.float32)] * 2,
        scratch_shapes=[
            pltpu.VMEM((_B, _D), jnp.float32),
            pltpu.VMEM((_B, _D), jnp.float32),
        ],
    )(h, batch2d, gf, mv, pw1, pb1, pw2, pb2, vw1, vb1, vw2, vb2)


def kernel(x, edge_index, edge_attr, batch, global_feats, move_feat,
           edge_W, edge_b, eps, mlp_W1, mlp_b1, mlp_W2, mlp_b2, ln_g, ln_b,
           pol_W1, pol_b1, pol_W2, pol_b2, val_W1, val_b1, val_W2, val_b2):
    src = edge_index[0]
    dst = edge_index[1]
    e_all = [_edge_mlp_one(edge_attr, edge_W[i], edge_b[i].reshape(1, _D))
             for i in range(_L)]

    h = x
    for i in range(_L):
        agg = _sc_msg(e_all[i], h, src, dst)
        h = _node_mlp(
            h, agg, eps[i].reshape(1, 1),
            mlp_W1[i], mlp_b1[i].reshape(1, _D),
            mlp_W2[i], mlp_b2[i].reshape(1, _D),
            ln_g[i].reshape(1, _D), ln_b[i].reshape(1, _D))

    pol, val = _pool_heads(
        h, batch.reshape(_N, 1), global_feats, move_feat,
        pol_W1, pol_b1.reshape(1, _D), pol_W2, pol_b2.reshape(1, 1),
        val_W1, val_b1.reshape(1, _D), val_W2, val_b2.reshape(1, 1))
    return (pol[:, 0], val[:, 0])


# speedup vs baseline: 1.0313x; 1.0313x over previous
import functools

import jax
import jax.numpy as jnp
from jax import lax
from jax.experimental import pallas as pl
from jax.experimental.pallas import tpu as pltpu
from jax.experimental.pallas import tpu_sc as plsc

_N = 10000
_E = 320000
_D = 128
_DE = 16
_B = 64
_G = 32
_MV = 5
_L = 3


_BE = 2000


def _edge_mlp_body(attr_ref, w_ref, b_ref, o_ref):
    a = attr_ref[...].astype(jnp.bfloat16)
    e = jnp.dot(a, w_ref[...].astype(jnp.bfloat16),
                preferred_element_type=jnp.float32)
    o_ref[...] = jnp.maximum(e + b_ref[...], 0.0)


def _edge_mlp_one(edge_attr, w, b):
    grid = (_E // _BE,)
    return pl.pallas_call(
        _edge_mlp_body,
        grid=grid,
        in_specs=[
            pl.BlockSpec((_BE, _DE), lambda i: (i, 0)),
            pl.BlockSpec((_DE, _D), lambda i: (0, 0)),
            pl.BlockSpec((1, _D), lambda i: (0, 0)),
        ],
        out_specs=pl.BlockSpec((_BE, _D), lambda i: (i, 0)),
        out_shape=jax.ShapeDtypeStruct((_E, _D), jnp.float32),
    )(edge_attr, w, b)


_NC = 2
_NS = 16
_NW = _NC * _NS
_EPT = _E // _NW
_K = 40
_CHUNKS = _EPT // _K
_NB = 2
_GROUPS = _CHUNKS // _NB
_NP = 10240
_RPT = _NP // _NS
_ZR = 128


def _sc_msg_body(e_hbm, h_hbm, src_hbm, dst_hbm, out_hbm,
                 idxs_s, idxs_d, bufs_e, bufs_h, zbuf, agg,
                 sems_i, sems_e, sems_g, sems_o):
    c = lax.axis_index("c")
    s = lax.axis_index("s")
    wid = c * _NS + s

    def _zrow(i, carry):
        for j in range(_D // 16):
            zbuf[i, pl.ds(j * 16, 16)] = jnp.zeros((16,), jnp.float32)
        return carry

    lax.fori_loop(0, _ZR, _zrow, 0)
    r0 = s * _RPT
    for z in range(_RPT // _ZR):
        pltpu.sync_copy(zbuf, agg.at[pl.ds(r0 + z * _ZR, _ZR)])
    plsc.subcore_barrier()

    ebase = wid * _EPT

    def _group(g, carry):
        cps = []
        for j in range(_NB):
            off = ebase + (g * _NB + j) * _K
            cp_s = pltpu.async_copy(src_hbm.at[pl.ds(off, _K)],
                                    idxs_s[j], sems_i[j])
            cp_d = pltpu.async_copy(dst_hbm.at[pl.ds(off, _K)],
                                    idxs_d[j], sems_i[j])
            cp_e = pltpu.async_copy(e_hbm.at[pl.ds(off, _K)],
                                    bufs_e[j], sems_e[j])
            cps.append((cp_s, cp_d, cp_e))
        gts = []
        for j in range(_NB):
            cps[j][0].wait()
            gts.append(pltpu.async_copy(h_hbm.at[idxs_s[j]], bufs_h[j],
                                        sems_g[j]))
        scs = []
        for j in range(_NB):
            cps[j][1].wait()
            cps[j][2].wait()
            gts[j].wait()

            def _row(i, carry2):
                r = i * 2
                for dr in range(2):
                    for q in range(_D // 16):
                        sl = pl.ds(q * 16, 16)
                        bufs_e[j][r + dr, sl] = jnp.maximum(
                            bufs_h[j][r + dr, sl] + bufs_e[j][r + dr, sl], 0.0)
                return carry2

            lax.fori_loop(0, _K // 2, _row, 0)
            scs.append(pltpu.async_copy(bufs_e[j], agg.at[idxs_d[j]],
                                        sems_o[j], add=True))
        for sc in scs:
            sc.wait()
        return carry

    lax.fori_loop(0, _GROUPS, _group, 0)
    plsc.subcore_barrier()

    for z in range(_RPT // _ZR):
        rr = s * _RPT + z * _ZR
        pltpu.sync_copy(agg.at[pl.ds(rr, _ZR)], out_hbm.at[c, pl.ds(rr, _ZR)])


@functools.cache
def _sc_msg_kernel():
    return pl.kernel(
        _sc_msg_body,
        out_type=jax.ShapeDtypeStruct((_NC, _NP, _D), jnp.float32),
        mesh=plsc.VectorSubcoreMesh(core_axis_name="c", subcore_axis_name="s"),
        scratch_types=[
            [pltpu.VMEM((_K,), jnp.int32)] * _NB,
            [pltpu.VMEM((_K,), jnp.int32)] * _NB,
            [pltpu.VMEM((_K, _D), jnp.float32)] * _NB,
            [pltpu.VMEM((_K, _D), jnp.float32)] * _NB,
            pltpu.VMEM((_ZR, _D), jnp.float32),
            pltpu.VMEM_SHARED((_NP, _D), jnp.float32),
            [pltpu.SemaphoreType.DMA] * _NB,
            [pltpu.SemaphoreType.DMA] * _NB,
            [pltpu.SemaphoreType.DMA] * _NB,
            [pltpu.SemaphoreType.DMA] * _NB,
        ],
    )


def _sc_msg(e, h, src, dst):
    return _sc_msg_kernel()(e, h, src, dst)


_BN = 1000


def _node_mlp_body(h_ref, agg_ref, eps_ref, w1_ref, b1_ref, w2_ref, b2_ref,
                   g_ref, bn_ref, out_ref):
    h = h_ref[...]
    t = (1.0 + eps_ref[0, 0]) * h + agg_ref[0] + agg_ref[1]
    u = jnp.dot(t.astype(jnp.bfloat16), w1_ref[...].astype(jnp.bfloat16),
                preferred_element_type=jnp.float32)
    u = jnp.maximum(u + b1_ref[...], 0.0)
    u = jnp.dot(u.astype(jnp.bfloat16), w2_ref[...].astype(jnp.bfloat16),
                preferred_element_type=jnp.float32)
    u = u + b2_ref[...]
    m = jnp.mean(u, axis=-1, keepdims=True)
    v = jnp.mean((u - m) * (u - m), axis=-1, keepdims=True)
    u = (u - m) * jax.lax.rsqrt(v + 1e-5) * g_ref[...] + bn_ref[...]
    out_ref[...] = jnp.maximum(u, 0.0) + h


def _node_mlp(h, agg, eps_i, w1, b1, w2, b2, g, bn):
    grid = (_N // _BN,)
    return pl.pallas_call(
        _node_mlp_body,
        grid=grid,
        in_specs=[
            pl.BlockSpec((_BN, _D), lambda i: (i, 0)),
            pl.BlockSpec((_NC, _BN, _D), lambda i: (0, i, 0)),
            pl.BlockSpec(memory_space=pltpu.SMEM),
            pl.BlockSpec((_D, _D), lambda i: (0, 0)),
            pl.BlockSpec((1, _D), lambda i: (0, 0)),
            pl.BlockSpec((_D, _D), lambda i: (0, 0)),
            pl.BlockSpec((1, _D), lambda i: (0, 0)),
            pl.BlockSpec((1, _D), lambda i: (0, 0)),
            pl.BlockSpec((1, _D), lambda i: (0, 0)),
        ],
        out_specs=pl.BlockSpec((_BN, _D), lambda i: (i, 0)),
        out_shape=jax.ShapeDtypeStruct((_N, _D), jnp.float32),
    )(h, agg, eps_i, w1, b1, w2, b2, g, bn)


def _pool_heads_body(h_ref, batch_ref, gf_ref, mv_ref,
                     pw1_ref, pb1_ref, pw2_ref, pb2_ref,
                     vw1_ref, vb1_ref, vw2_ref, vb2_ref,
                     pol_ref, val_ref, pooled_acc, cnt_acc):
    i = pl.program_id(0)

    @pl.when(i == 0)
    def _init():
        pooled_acc[...] = jnp.zeros_like(pooled_acc)
        cnt_acc[...] = jnp.zeros_like(cnt_acc)

    h = h_ref[...]
    b = batch_ref[...]
    oh = (b == lax.broadcasted_iota(jnp.int32, (_BN, _B), 1)).astype(jnp.float32)
    pooled_acc[...] += lax.dot_general(
        oh, h, (((0,), (0,)), ((), ())), preferred_element_type=jnp.float32,
        precision=jax.lax.Precision.HIGHEST)
    cnt_acc[...] += lax.dot_general(
        oh, jnp.ones((_BN, _D), jnp.float32), (((0,), (0,)), ((), ())),
        preferred_element_type=jnp.float32,
        precision=jax.lax.Precision.HIGHEST)

    @pl.when(i == pl.num_programs(0) - 1)
    def _heads():
        pooled = pooled_acc[...] / jnp.maximum(cnt_acc[...], 1.0)
        v_in = jnp.concatenate([pooled, gf_ref[...]], axis=1)
        v = jnp.maximum(
            jnp.dot(v_in.astype(jnp.bfloat16), vw1_ref[...].astype(jnp.bfloat16),
                    preferred_element_type=jnp.float32)
            + vb1_ref[...], 0.0)
        val_ref[...] = (jnp.dot(v.astype(jnp.bfloat16),
                                vw2_ref[...].astype(jnp.bfloat16),
                                preferred_element_type=jnp.float32)
                        + vb2_ref[...])
        p_in = jnp.concatenate([pooled, mv_ref[...]], axis=1)
        p = jnp.maximum(
            jnp.dot(p_in.astype(jnp.bfloat16), pw1_ref[...].astype(jnp.bfloat16),
                    preferred_element_type=jnp.float32)
            + pb1_ref[...], 0.0)
        pol_ref[...] = (jnp.dot(p.astype(jnp.bfloat16),
                                pw2_ref[...].astype(jnp.bfloat16),
                                preferred_element_type=jnp.float32)
                        + pb2_ref[...])


def _pool_heads(h, batch2d, gf, mv, pw1, pb1, pw2, pb2, vw1, vb1, vw2, vb2):
    grid = (_N // _BN,)
    full = lambda shape: pl.BlockSpec(shape, lambda i: tuple(0 for _ in shape))
    return pl.pallas_call(
        _pool_heads_body,
        grid=grid,
        in_specs=[
            pl.BlockSpec((_BN, _D), lambda i: (i, 0)),
            pl.BlockSpec((_BN, 1), lambda i: (i, 0)),
            full((_B, _G)),
            full((_B, _MV)),
            full((_D + _MV, _D)),
            full((1, _D)),
            full((_D, 1)),
            full((1, 1)),
            full((_D + _G, _D)),
            full((1, _D)),
            full((_D, 1)),
            full((1, 1)),
        ],
        out_specs=[full((_B, 1)), full((_B, 1))],
        out_shape=[jax.ShapeDtypeStruct((_B, 1), jnp.float32)] * 2,
        scratch_shapes=[
            pltpu.VMEM((_B, _D), jnp.float32),
            pltpu.VMEM((_B, _D), jnp.float32),
        ],
    )(h, batch2d, gf, mv, pw1, pb1, pw2, pb2, vw1, vb1, vw2, vb2)


def kernel(x, edge_index, edge_attr, batch, global_feats, move_feat,
           edge_W, edge_b, eps, mlp_W1, mlp_b1, mlp_W2, mlp_b2, ln_g, ln_b,
           pol_W1, pol_b1, pol_W2, pol_b2, val_W1, val_b1, val_W2, val_b2):
    src = edge_index[0]
    dst = edge_index[1]
    e_all = [_edge_mlp_one(edge_attr, edge_W[i], edge_b[i].reshape(1, _D))
             for i in range(_L)]

    h = x
    for i in range(_L):
        agg = _sc_msg(e_all[i], h, src, dst)
        h = _node_mlp(
            h, agg, eps[i].reshape(1, 1),
            mlp_W1[i], mlp_b1[i].reshape(1, _D),
            mlp_W2[i], mlp_b2[i].reshape(1, _D),
            ln_g[i].reshape(1, _D), ln_b[i].reshape(1, _D))

    pol, val = _pool_heads(
        h, batch.reshape(_N, 1), global_feats, move_feat,
        pol_W1, pol_b1.reshape(1, _D), pol_W2, pol_b2.reshape(1, 1),
        val_W1, val_b1.reshape(1, _D), val_W2, val_b2.reshape(1, 1))
    return (pol[:, 0], val[:, 0])
